# baseline (speedup 1.0000x reference)
import functools

import jax
import jax.numpy as jnp
from jax import lax
from jax.experimental import pallas as pl
from jax.experimental.pallas import tpu as pltpu
from jax.experimental.pallas import tpu_sc as plsc

_R, _S = 16384, 26
_V, _D = 10000, 64
_N = _R * _S
_NC, _NS = 2, 16
_NW = _NC * _NS
_IPW = _R // _NW
_N_PAIRS = _S // 2

_SD = _S * _D
_B1 = 2048


def _sc_gather(idx_t, table):
    mesh = plsc.VectorSubcoreMesh(core_axis_name="c", subcore_axis_name="s")

    @functools.partial(
        pl.kernel,
        mesh=mesh,
        out_type=jax.ShapeDtypeStruct((_N, _D), jnp.float32),
        scratch_types=[
            pltpu.VMEM((_S, _IPW), jnp.int32),
            pltpu.VMEM((_IPW, _D), jnp.float32),
            pltpu.VMEM((_IPW, _D), jnp.float32),
            pltpu.SemaphoreType.DMA,
            pltpu.SemaphoreType.DMA,
            pltpu.SemaphoreType.DMA,
            pltpu.SemaphoreType.DMA,
        ],
        compiler_params=pltpu.CompilerParams(use_tc_tiling_on_sc=False),
    )
    def k(idx_hbm, table_hbm, out_hbm, idx_all, rows0, rows1, sg0, sg1, sw0, sw1):
        wid = lax.axis_index("s") * _NC + lax.axis_index("c")
        base = wid * _IPW
        pltpu.sync_copy(idx_hbm.at[pl.ds(0, _S), pl.ds(base, _IPW)], idx_all)

        def gather(j, buf, sem):
            pltpu.async_copy(table_hbm.at[idx_all.at[j]], buf, sem)

        def wait_gather(j, buf, sem):
            pltpu.make_async_copy(table_hbm.at[idx_all.at[j]], buf, sem).wait()

        def write(j, buf, sem):
            pltpu.async_copy(buf, out_hbm.at[pl.ds(j * _R + base, _IPW)], sem)

        def wait_write(j, buf, sem):
            pltpu.make_async_copy(
                buf, out_hbm.at[pl.ds(j * _R + base, _IPW)], sem
            ).wait()

        gather(0, rows0, sg0)

        def body(g, carry):
            c0 = 2 * g
            c1 = c0 + 1
            wait_gather(c0, rows0, sg0)
            write(c0, rows0, sw0)

            @pl.when(g > 0)
            def _():
                wait_write(c0 - 1, rows1, sw1)

            gather(c1, rows1, sg1)
            wait_gather(c1, rows1, sg1)
            write(c1, rows1, sw1)
            wait_write(c0, rows0, sw0)

            @pl.when(g < _N_PAIRS - 1)
            def _():
                gather(c0 + 2, rows0, sg0)

            return carry

        lax.fori_loop(0, _N_PAIRS, body, 0)
        wait_write(_S - 1, rows1, sw1)

    return k(idx_t, table)


def _tc_transpose_kernel(x_ref, y_ref):
    xt = x_ref[...].T
    a = xt[: _D, :]
    b = xt[_D :, :]
    y_ref[...] = jnp.stack([a, b], axis=-1).reshape(_D, _B1)


def _tc_transpose(x2):
    return pl.pallas_call(
        _tc_transpose_kernel,
        grid=(_S, _R // _B1),
        in_specs=[
            pl.BlockSpec(
                (_B1 // 2, 128), lambda j, b: (j * (_R // _B1) + b, 0)
            )
        ],
        out_specs=pl.BlockSpec((_D, _B1), lambda j, b: (j, b)),
        out_shape=jax.ShapeDtypeStruct((_SD, _R), jnp.float32),
    )(x2)


def kernel(data, embeddings):
    idx_t = data.T
    flat = _sc_gather(idx_t, embeddings)
    y = _tc_transpose(flat.reshape(_N // 2, 2 * _D))
    return jnp.transpose(y.reshape(_S, _D, _R), (2, 0, 1))


# speedup vs baseline: 26.5928x; 26.5928x over previous
import functools

import jax
import jax.numpy as jnp
from jax import lax
from jax.experimental import pallas as pl
from jax.experimental.pallas import tpu as pltpu
from jax.experimental.pallas import tpu_sc as plsc

_R, _S = 16384, 26
_V, _D = 10000, 64
_N = _R * _S
_H = _R // 2
_NC, _NS = 2, 16
_NW = _NC * _NS
_IPW = _R // _NW
_N_PAIRS = _S // 2

_SD = _S * _D
_B1 = 2048


def _sc_gather(idx_t, table):
    mesh = plsc.VectorSubcoreMesh(core_axis_name="c", subcore_axis_name="s")

    @functools.partial(
        pl.kernel,
        mesh=mesh,
        out_type=jax.ShapeDtypeStruct((_N // 2, 2 * _D), jnp.float32),
        scratch_types=[
            pltpu.VMEM((_S, _IPW), jnp.int32),
            pltpu.VMEM((_IPW, _D), jnp.float32),
            pltpu.VMEM((_IPW, _D), jnp.float32),
            pltpu.SemaphoreType.DMA,
            pltpu.SemaphoreType.DMA,
            pltpu.SemaphoreType.DMA,
            pltpu.SemaphoreType.DMA,
        ],
        compiler_params=pltpu.CompilerParams(use_tc_tiling_on_sc=False),
    )
    def k(idx_hbm, table_hbm, out_hbm, idx_all, rows0, rows1, sg0, sg1, sw0, sw1):
        wid = lax.axis_index("s") * _NC + lax.axis_index("c")
        base = wid * _IPW
        pltpu.sync_copy(idx_hbm.at[pl.ds(0, _S), pl.ds(base, _IPW)], idx_all)

        def gather(j, buf, sem):
            pltpu.async_copy(table_hbm.at[idx_all.at[j]], buf, sem)

        def wait_gather(j, buf, sem):
            pltpu.make_async_copy(table_hbm.at[idx_all.at[j]], buf, sem).wait()

        def write(j, buf, sem):
            pltpu.async_copy(
                buf,
                out_hbm.at[
                    pl.ds((j // 2) * _R + base, _IPW), pl.ds((j % 2) * _D, _D)
                ],
                sem,
            )

        def wait_write(j, buf, sem):
            pltpu.make_async_copy(
                buf,
                out_hbm.at[
                    pl.ds((j // 2) * _R + base, _IPW), pl.ds((j % 2) * _D, _D)
                ],
                sem,
            ).wait()

        gather(0, rows0, sg0)

        def body(g, carry):
            c0 = 2 * g
            c1 = c0 + 1
            wait_gather(c0, rows0, sg0)
            write(c0, rows0, sw0)

            @pl.when(g > 0)
            def _():
                wait_write(c0 - 1, rows1, sw1)

            gather(c1, rows1, sg1)
            wait_gather(c1, rows1, sg1)
            write(c1, rows1, sw1)
            wait_write(c0, rows0, sw0)

            @pl.when(g < _N_PAIRS - 1)
            def _():
                gather(c0 + 2, rows0, sg0)

            return carry

        lax.fori_loop(0, _N_PAIRS, body, 0)
        wait_write(_S - 1, rows1, sw1)

    return k(idx_t, table)


def _tc_transpose_kernel(x_ref, y_ref):
    y_ref[...] = x_ref[...].T


def _tc_transpose(x2):
    nb = _R // _B1
    return pl.pallas_call(
        _tc_transpose_kernel,
        grid=(_S // 2, nb),
        in_specs=[pl.BlockSpec((_B1, 128), lambda u, b: (u * nb + b, 0))],
        out_specs=pl.BlockSpec((128, _B1), lambda u, b: (u, b)),
        out_shape=jax.ShapeDtypeStruct((_SD, _R), jnp.float32),
    )(x2)


def kernel(data, embeddings):
    idx_t = data.T
    flat = _sc_gather(idx_t, embeddings)
    y = _tc_transpose(flat)
    return jnp.transpose(y.reshape(_S, _D, _R), (2, 0, 1))
